# baseline (speedup 1.0000x reference)
import functools

import jax
import jax.numpy as jnp
import numpy as np
from jax import lax
from jax.experimental import pallas as pl
from jax.experimental.pallas import tpu as pltpu
from jax.experimental.pallas import tpu_sc as plsc

N = 10000
E = 160000
HIDDEN = 4096
EMBED = 256
Dh = 128

NS = 16
EPT = E // NS
B = 80
NCHUNK = EPT // B
NACC = 10240
RPT = NACC // NS

_mesh = plsc.VectorSubcoreMesh(core_axis_name="c", subcore_axis_name="s",
                               num_cores=2, num_subcores=NS)


@functools.partial(
    pl.kernel,
    out_type=jax.ShapeDtypeStruct((2, NACC, Dh), jnp.float32),
    mesh=_mesh,
    scratch_types=[
        [pltpu.VMEM((B,), jnp.int32)] * 2,
        [pltpu.VMEM((B,), jnp.int32)] * 2,
        [pltpu.VMEM((B,), jnp.float32)] * 2,
        [pltpu.VMEM((B, Dh), jnp.bfloat16)] * 2,
        [pltpu.VMEM((B, Dh), jnp.float32)] * 2,
        [pltpu.VMEM((B,), jnp.int32)] * 2,
        pltpu.VMEM((Dh,), jnp.float32),
        pltpu.VMEM((Dh,), jnp.float32),
        pltpu.VMEM_SHARED((NACC, Dh), jnp.float32),
        [pltpu.SemaphoreType.DMA] * 2,
        [pltpu.SemaphoreType.DMA] * 2,
        [pltpu.SemaphoreType.DMA] * 2,
        [pltpu.SemaphoreType.DMA] * 2,
    ],
    compiler_params=pltpu.CompilerParams(use_tc_tiling_on_sc=False,
                                         needs_layout_passes=False),
)
def _sc_aggr(table, src, dst, attr, WeH, beH, zrows, out,
             gi, sb, ab, rows, msg, di, wev, bev, aggr,
             sem_i, sem_d, sem_g, sem_s):
    c = lax.axis_index("c")
    s = lax.axis_index("s")
    pltpu.sync_copy(WeH.at[c], wev)
    pltpu.sync_copy(beH.at[c], bev)
    pltpu.sync_copy(zrows, aggr.at[pl.ds(s * RPT, RPT)])
    plsc.subcore_barrier()

    we = [wev[pl.ds(16 * j, 16)] for j in range(Dh // 16)]
    be = [bev[pl.ds(16 * j, 16)] for j in range(Dh // 16)]

    ebase = s * EPT

    def fire_in(k, b):
        eb = ebase + k * B
        pltpu.async_copy(src.at[pl.ds(eb, B)], sb[b], sem_i[b])
        pltpu.async_copy(attr.at[pl.ds(eb, B)], ab[b], sem_i[b])

    def wait_in(k, b):
        eb = ebase + k * B
        pltpu.make_async_copy(src.at[pl.ds(eb, B)], sb[b], sem_i[b]).wait()
        pltpu.make_async_copy(attr.at[pl.ds(eb, B)], ab[b], sem_i[b]).wait()

    def fire_di(k, b):
        pltpu.async_copy(dst.at[pl.ds(ebase + k * B, B)], di[b], sem_d[b])

    def wait_di(k, b):
        pltpu.make_async_copy(dst.at[pl.ds(ebase + k * B, B)], di[b],
                              sem_d[b]).wait()

    def transform(b):
        for j in range(B // 16):
            v = sb[b][pl.ds(16 * j, 16)]
            gi[b][pl.ds(16 * j, 16)] = v * 2 + c

    def fire_gather(b):
        pltpu.async_copy(table.at[gi[b]], rows[b], sem_g[b])

    def wait_gather(b):
        pltpu.make_async_copy(table.at[gi[b]], rows[b], sem_g[b]).wait()

    def fire_scatter(b):
        pltpu.async_copy(msg[b], aggr.at[di[b]], sem_s[b], add=True)

    def wait_scatter(b):
        pltpu.make_async_copy(msg[b], aggr.at[di[b]], sem_s[b]).wait()

    def compute(b):
        def grp(g, _):
            a_vec = ab[b][pl.ds(16 * g, 16)]
            for i in range(16):
                a = a_vec[i]
                e = 16 * g + i
                for j in range(Dh // 32):
                    v32 = rows[b][e, pl.ds(32 * j, 32)]
                    va, vb = plsc.unpack(v32,
                                         format=plsc.PackFormat.INTERLEAVED)
                    msg[b][e, pl.ds(32 * j, 16)] = jnp.maximum(
                        va + (a * we[2 * j] + be[2 * j]), 0.0)
                    msg[b][e, pl.ds(32 * j + 16, 16)] = jnp.maximum(
                        vb + (a * we[2 * j + 1] + be[2 * j + 1]), 0.0)
            return 0

        lax.fori_loop(0, B // 16, grp, 0)

    fire_in(0, 0)
    fire_in(1, 1)
    fire_di(0, 0)
    wait_in(0, 0)
    transform(0)
    fire_gather(0)

    def step(k, b):
        bo = 1 - b
        wait_gather(b)
        wait_in(k + 1, bo)
        transform(bo)

        @pl.when(k >= 1)
        def _():
            wait_scatter(bo)

        fire_di(k + 1, bo)
        fire_gather(bo)
        wait_di(k, b)
        compute(b)
        fire_scatter(b)

        @pl.when(k + 2 < NCHUNK)
        def _():
            fire_in(k + 2, b)

    def outer(K, _):
        step(2 * K, 0)
        step(2 * K + 1, 1)
        return 0

    lax.fori_loop(0, (NCHUNK - 1) // 2, outer, 0)

    wait_gather(0)
    wait_di(NCHUNK - 1, 0)
    compute(0)
    fire_scatter(0)
    wait_scatter(1)
    wait_scatter(0)

    plsc.subcore_barrier()
    pltpu.sync_copy(aggr.at[pl.ds(s * RPT, RPT)],
                    out.at[c, pl.ds(s * RPT, RPT)])


@functools.partial(
    pl.kernel,
    out_type=jax.ShapeDtypeStruct((NACC,), jnp.float32),
    mesh=_mesh,
    scratch_types=[
        pltpu.VMEM((EPT,), jnp.int32),
        pltpu.VMEM((EPT,), jnp.int32),
        pltpu.VMEM((EPT,), jnp.float32),
        pltpu.VMEM((NACC,), jnp.float32),
        pltpu.VMEM((NACC,), jnp.float32),
        pltpu.VMEM((16, RPT), jnp.float32),
        pltpu.VMEM((RPT,), jnp.float32),
        pltpu.VMEM((16,), jnp.float32),
        pltpu.VMEM((16,), jnp.float32),
        pltpu.VMEM_SHARED((16, NACC), jnp.float32),
        pltpu.SemaphoreType.DMA,
    ],
    compiler_params=pltpu.CompilerParams(needs_layout_passes=False),
)
def _sc_extra(rs, src, dst, attr, wxH, bxH, out,
              gi, di, av, rsv, acc, red16, red, wxv, bxv, acc_sh, sem):
    c = lax.axis_index("c")
    s = lax.axis_index("s")

    @pl.when(c == 0)
    def _():
        pltpu.sync_copy(rs, rsv)
        pltpu.sync_copy(wxH, wxv)
        pltpu.sync_copy(bxH, bxv)
        pltpu.sync_copy(src.at[pl.ds(s * EPT, EPT)], gi)
        pltpu.sync_copy(dst.at[pl.ds(s * EPT, EPT)], di)
        pltpu.sync_copy(attr.at[pl.ds(s * EPT, EPT)], av)
        wx = wxv[...]
        bx = bxv[...]
        zero = jnp.zeros((16,), jnp.float32)
        def z(j, _):
            acc[pl.ds(16 * j, 16)] = zero
            return 0
        lax.fori_loop(0, NACC // 16, z, 0)

        def group(g, _):
            s16 = gi[pl.ds(16 * g, 16)]
            d16 = di[pl.ds(16 * g, 16)]
            a16 = av[pl.ds(16 * g, 16)]
            vals = plsc.load_gather(rsv, [s16])
            msg = jnp.maximum(vals + (a16 * wx + bx), 0.0)
            plsc.addupdate_scatter(acc, [d16], msg)
            return 0

        lax.fori_loop(0, EPT // 16, group, 0)

        pltpu.sync_copy(acc, acc_sh.at[s])
        plsc.subcore_barrier()
        pltpu.sync_copy(acc_sh.at[:, pl.ds(s * RPT, RPT)], red16)

        def radd(j, _):
            t = red16[0, pl.ds(16 * j, 16)]
            for i in range(1, 16):
                t = t + red16[i, pl.ds(16 * j, 16)]
            red[pl.ds(16 * j, 16)] = t
            return 0

        lax.fori_loop(0, RPT // 16, radd, 0)
        pltpu.sync_copy(red, out.at[pl.ds(s * RPT, RPT)])


def _make_mlp(first):
    R = 2000
    G = N // R
    D = 256
    KD = 264 if first else 256

    def body(h_ref, a0_ref, a1_ref, w1_ref, w1s_ref, b1_ref, w2_ref, b2_ref,
             *rest):
        if first:
            xcol_ref, out_ref, outbf_ref = rest
        else:
            xp_ref, out_ref, outbf_ref = rest
        z = h_ref[...].astype(jnp.bfloat16)
        if first:
            xcb = jnp.concatenate(
                [xcol_ref[...], jnp.zeros((R, 7), jnp.float32)],
                axis=1).astype(jnp.bfloat16)
            z = jnp.concatenate([z, xcb], axis=1)
        za = jnp.concatenate([a0_ref[...], a1_ref[...]],
                             axis=1).astype(jnp.bfloat16)
        hh = jnp.dot(z, w1_ref[...], preferred_element_type=jnp.float32)
        hh += jnp.dot(za, w1s_ref[...], preferred_element_type=jnp.float32)
        hh = jnp.maximum(hh + b1_ref[...], 0.0).astype(jnp.bfloat16)
        o = jnp.dot(hh, w2_ref[...], preferred_element_type=jnp.float32) + b2_ref[...]
        if not first:
            o = (jnp.where(o > 0, o, 0.01 * o) + xp_ref[...]) * 0.5
        out_ref[...] = o
        outbf_ref[...] = o.astype(jnp.bfloat16)

    in_specs = [
        pl.BlockSpec((R, D), lambda i: (i, 0)),
        pl.BlockSpec((R, Dh), lambda i: (i, 0)),
        pl.BlockSpec((R, Dh), lambda i: (i, 0)),
        pl.BlockSpec((KD, HIDDEN), lambda i: (0, 0)),
        pl.BlockSpec((D, HIDDEN), lambda i: (0, 0)),
        pl.BlockSpec((1, HIDDEN), lambda i: (0, 0)),
        pl.BlockSpec((HIDDEN, EMBED), lambda i: (0, 0)),
        pl.BlockSpec((1, EMBED), lambda i: (0, 0)),
    ]
    if first:
        in_specs.append(pl.BlockSpec((R, 1), lambda i: (i, 0)))
    else:
        in_specs.append(pl.BlockSpec((R, EMBED), lambda i: (i, 0)))

    return pl.pallas_call(
        body,
        grid=(G,),
        in_specs=in_specs,
        out_specs=[pl.BlockSpec((R, EMBED), lambda i: (i, 0)),
                   pl.BlockSpec((R, EMBED), lambda i: (i, 0))],
        out_shape=[jax.ShapeDtypeStruct((N, EMBED), jnp.float32),
                   jax.ShapeDtypeStruct((N, EMBED), jnp.bfloat16)],
    )


_mlp_first = _make_mlp(first=True)
_mlp_res = _make_mlp(first=False)


def kernel(x, edge_index, edge_attr,
           l0_We, l0_be, l0_W1, l0_b1, l0_W2, l0_b2,
           l1_We, l1_be, l1_W1, l1_b1, l1_W2, l1_b2,
           l2_We, l2_be, l2_W1, l2_b1, l2_W2, l2_b2):
    rs = jax.random.normal(jax.random.key(42), (N, 1), dtype=x.dtype)

    src = edge_index[0]
    dst = edge_index[1]
    attr = edge_attr[:, 0]
    zrows = jnp.zeros((RPT, Dh), jnp.float32)
    sperm = np.concatenate(
        [b * 32 + np.concatenate([np.arange(0, 32, 2), np.arange(1, 32, 2)])
         for b in range(8)])

    WeH = l0_We[0, :256][sperm].reshape(2, Dh)
    beH = l0_be[:256][sperm].reshape(2, Dh)
    wx = jnp.full((16,), l0_We[0, 256], jnp.float32)
    bx = jnp.full((16,), l0_be[256], jnp.float32)
    rs_pad = jnp.concatenate([rs[:, 0], jnp.zeros((NACC - N,), jnp.float32)])

    aggr = _sc_aggr(x.astype(jnp.bfloat16).reshape(2 * N, Dh), src, dst,
                    attr, WeH, beH, zrows)
    aggr_x = _sc_extra(rs_pad, src, dst, attr, wx, bx)
    xcol = rs + aggr_x[:N, None]

    W1p = jnp.concatenate(
        [l0_W1, jnp.zeros((7, HIDDEN), l0_W1.dtype)]).astype(jnp.bfloat16)
    W1s = l0_W1[:256][sperm].astype(jnp.bfloat16)
    xprev, xprev_bf = _mlp_first(x, aggr[0], aggr[1],
                                 W1p, W1s, l0_b1.reshape(1, HIDDEN),
                                 l0_W2.astype(jnp.bfloat16),
                                 l0_b2.reshape(1, EMBED), xcol)

    for (We, be, W1, b1v, W2, b2v) in (
            (l1_We, l1_be, l1_W1, l1_b1, l1_W2, l1_b2),
            (l2_We, l2_be, l2_W1, l2_b1, l2_W2, l2_b2)):
        WeH = We[0][sperm].reshape(2, Dh)
        beH = be[sperm].reshape(2, Dh)
        aggr = _sc_aggr(xprev_bf.reshape(2 * N, Dh), src, dst,
                        attr, WeH, beH, zrows)
        xprev, xprev_bf = _mlp_res(xprev, aggr[0], aggr[1],
                                   W1.astype(jnp.bfloat16),
                                   W1[sperm].astype(jnp.bfloat16),
                                   b1v.reshape(1, HIDDEN),
                                   W2.astype(jnp.bfloat16),
                                   b2v.reshape(1, EMBED), xprev)
    return xprev


# speedup vs baseline: 1.7939x; 1.7939x over previous
import functools

import jax
import jax.numpy as jnp
import numpy as np
from jax import lax
from jax.experimental import pallas as pl
from jax.experimental.pallas import tpu as pltpu
from jax.experimental.pallas import tpu_sc as plsc

N = 10000
E = 160000
HIDDEN = 4096
EMBED = 256
Dh = 128

NS = 16
EPT = E // NS
B = 80
NCHUNK = EPT // B
NACC = 10240
RPT = NACC // NS

_mesh = plsc.VectorSubcoreMesh(core_axis_name="c", subcore_axis_name="s",
                               num_cores=2, num_subcores=NS)


@functools.partial(
    pl.kernel,
    out_type=jax.ShapeDtypeStruct((2, NACC, Dh), jnp.float32),
    mesh=_mesh,
    scratch_types=[
        [pltpu.VMEM((B,), jnp.int32)] * 2,
        [pltpu.VMEM((B,), jnp.int32)] * 2,
        [pltpu.VMEM((B,), jnp.float32)] * 2,
        [pltpu.VMEM((B, Dh), jnp.float32)] * 2,
        [pltpu.VMEM((B,), jnp.int32)] * 2,
        pltpu.VMEM((1, EPT), jnp.int32),
        pltpu.VMEM((Dh,), jnp.float32),
        pltpu.VMEM((Dh,), jnp.float32),
        pltpu.VMEM_SHARED((NACC, Dh), jnp.float32),
        [pltpu.SemaphoreType.DMA] * 2,
        [pltpu.SemaphoreType.DMA] * 2,
        [pltpu.SemaphoreType.DMA] * 2,
    ],
)
def _sc_aggr(table, src, dst3, attr, WeH, beH, zrows, out,
             gi, sb, ab, rows, di, dall, wev, bev, aggr,
             sem_i, sem_g, sem_s):
    c = lax.axis_index("c")
    s = lax.axis_index("s")
    pltpu.sync_copy(WeH.at[c], wev)
    pltpu.sync_copy(beH.at[c], bev)
    pltpu.sync_copy(zrows, aggr.at[pl.ds(s * RPT, RPT)])
    plsc.subcore_barrier()
    pltpu.sync_copy(dst3.at[s], dall)

    we = [wev[pl.ds(16 * j, 16)] for j in range(Dh // 16)]
    be = [bev[pl.ds(16 * j, 16)] for j in range(Dh // 16)]

    ebase = s * EPT

    def fire_in(k, b):
        eb = ebase + k * B
        pltpu.async_copy(src.at[pl.ds(eb, B)], sb[b], sem_i[b])
        pltpu.async_copy(attr.at[pl.ds(eb, B)], ab[b], sem_i[b])

    def wait_in(k, b):
        eb = ebase + k * B
        pltpu.make_async_copy(src.at[pl.ds(eb, B)], sb[b], sem_i[b]).wait()
        pltpu.make_async_copy(attr.at[pl.ds(eb, B)], ab[b], sem_i[b]).wait()

    def transform(b):
        for j in range(B // 16):
            v = sb[b][pl.ds(16 * j, 16)]
            gi[b][pl.ds(16 * j, 16)] = v * 2 + c

    def fire_gather(b):
        pltpu.async_copy(table.at[gi[b]], rows[b], sem_g[b])

    def wait_gather(b):
        pltpu.make_async_copy(table.at[gi[b]], rows[b], sem_g[b]).wait()

    def fire_scatter(b):
        pltpu.async_copy(rows[b], aggr.at[di[b]], sem_s[b], add=True)

    def wait_scatter(b):
        pltpu.make_async_copy(rows[b], aggr.at[di[b]], sem_s[b]).wait()

    def fill_di(k, b):
        for j in range(B // 16):
            di[b][pl.ds(16 * j, 16)] = dall[0, pl.ds(k * B + 16 * j, 16)]

    def compute(b):
        def grp(g, _):
            a_vec = ab[b][pl.ds(16 * g, 16)]
            for i in range(16):
                a = a_vec[i]
                e = 16 * g + i
                for j in range(Dh // 16):
                    x = rows[b][e, pl.ds(16 * j, 16)]
                    rows[b][e, pl.ds(16 * j, 16)] = jnp.maximum(
                        x + (a * we[j] + be[j]), 0.0)
            return 0

        lax.fori_loop(0, B // 16, grp, 0)

    fire_in(0, 0)
    fire_in(1, 1)
    wait_in(0, 0)
    transform(0)
    fire_gather(0)

    def step(k, b):
        bo = 1 - b
        wait_gather(b)
        wait_in(k + 1, bo)
        transform(bo)

        @pl.when(k >= 1)
        def _():
            wait_scatter(bo)

        fire_gather(bo)
        fill_di(k, b)
        compute(b)
        fire_scatter(b)

        @pl.when(k + 2 < NCHUNK)
        def _():
            fire_in(k + 2, b)

    def outer(K, _):
        step(2 * K, 0)
        step(2 * K + 1, 1)
        return 0

    lax.fori_loop(0, (NCHUNK - 1) // 2, outer, 0)

    wait_gather(0)
    fill_di(NCHUNK - 1, 0)
    compute(0)
    fire_scatter(0)
    wait_scatter(1)
    wait_scatter(0)

    plsc.subcore_barrier()
    pltpu.sync_copy(aggr.at[pl.ds(s * RPT, RPT)],
                    out.at[c, pl.ds(s * RPT, RPT)])


@functools.partial(
    pl.kernel,
    out_type=jax.ShapeDtypeStruct((NACC,), jnp.float32),
    mesh=_mesh,
    scratch_types=[
        pltpu.VMEM((EPT,), jnp.int32),
        pltpu.VMEM((EPT,), jnp.int32),
        pltpu.VMEM((EPT,), jnp.float32),
        pltpu.VMEM((NACC,), jnp.float32),
        pltpu.VMEM((NACC,), jnp.float32),
        pltpu.VMEM((16, RPT), jnp.float32),
        pltpu.VMEM((RPT,), jnp.float32),
        pltpu.VMEM((16,), jnp.float32),
        pltpu.VMEM((16,), jnp.float32),
        pltpu.VMEM_SHARED((16, NACC), jnp.float32),
        pltpu.SemaphoreType.DMA,
    ],
    compiler_params=pltpu.CompilerParams(needs_layout_passes=False),
)
def _sc_extra(rs, src, dst, attr, wxH, bxH, out,
              gi, di, av, rsv, acc, red16, red, wxv, bxv, acc_sh, sem):
    c = lax.axis_index("c")
    s = lax.axis_index("s")

    @pl.when(c == 0)
    def _():
        pltpu.sync_copy(rs, rsv)
        pltpu.sync_copy(wxH, wxv)
        pltpu.sync_copy(bxH, bxv)
        pltpu.sync_copy(src.at[pl.ds(s * EPT, EPT)], gi)
        pltpu.sync_copy(dst.at[pl.ds(s * EPT, EPT)], di)
        pltpu.sync_copy(attr.at[pl.ds(s * EPT, EPT)], av)
        wx = wxv[...]
        bx = bxv[...]
        zero = jnp.zeros((16,), jnp.float32)
        def z(j, _):
            acc[pl.ds(16 * j, 16)] = zero
            return 0
        lax.fori_loop(0, NACC // 16, z, 0)

        def group(g, _):
            s16 = gi[pl.ds(16 * g, 16)]
            d16 = di[pl.ds(16 * g, 16)]
            a16 = av[pl.ds(16 * g, 16)]
            vals = plsc.load_gather(rsv, [s16])
            msg = jnp.maximum(vals + (a16 * wx + bx), 0.0)
            plsc.addupdate_scatter(acc, [d16], msg)
            return 0

        lax.fori_loop(0, EPT // 16, group, 0)

        pltpu.sync_copy(acc, acc_sh.at[s])
        plsc.subcore_barrier()
        pltpu.sync_copy(acc_sh.at[:, pl.ds(s * RPT, RPT)], red16)

        def radd(j, _):
            t = red16[0, pl.ds(16 * j, 16)]
            for i in range(1, 16):
                t = t + red16[i, pl.ds(16 * j, 16)]
            red[pl.ds(16 * j, 16)] = t
            return 0

        lax.fori_loop(0, RPT // 16, radd, 0)
        pltpu.sync_copy(red, out.at[pl.ds(s * RPT, RPT)])


def _make_mlp(first):
    R = 1000
    G = N // R
    D = 256
    KD = 264 if first else 256

    def body(h_ref, a0_ref, a1_ref, w1_ref, b1_ref, w2_ref, b2_ref,
             *rest):
        if first:
            xcol_ref, out_ref = rest
        else:
            xp_ref, out_ref = rest
        za = jnp.concatenate([a0_ref[...], a1_ref[...]], axis=1)
        z = (h_ref[...] + za).astype(jnp.bfloat16)
        if first:
            xcb = jnp.concatenate(
                [xcol_ref[...], jnp.zeros((R, 7), jnp.float32)],
                axis=1).astype(jnp.bfloat16)
            z = jnp.concatenate([z, xcb], axis=1)
        hh = jnp.dot(z, w1_ref[...], preferred_element_type=jnp.float32)
        hh = jnp.maximum(hh + b1_ref[...], 0.0).astype(jnp.bfloat16)
        o = jnp.dot(hh, w2_ref[...], preferred_element_type=jnp.float32) + b2_ref[...]
        if not first:
            o = (jnp.where(o > 0, o, 0.01 * o) + xp_ref[...]) * 0.5
        out_ref[...] = o

    in_specs = [
        pl.BlockSpec((R, D), lambda i: (i, 0)),
        pl.BlockSpec((R, Dh), lambda i: (i, 0)),
        pl.BlockSpec((R, Dh), lambda i: (i, 0)),
        pl.BlockSpec((KD, HIDDEN), lambda i: (0, 0)),
        pl.BlockSpec((1, HIDDEN), lambda i: (0, 0)),
        pl.BlockSpec((HIDDEN, EMBED), lambda i: (0, 0)),
        pl.BlockSpec((1, EMBED), lambda i: (0, 0)),
    ]
    if first:
        in_specs.append(pl.BlockSpec((R, 1), lambda i: (i, 0)))
    else:
        in_specs.append(pl.BlockSpec((R, EMBED), lambda i: (i, 0)))

    return pl.pallas_call(
        body,
        grid=(G,),
        in_specs=in_specs,
        out_specs=pl.BlockSpec((R, EMBED), lambda i: (i, 0)),
        out_shape=jax.ShapeDtypeStruct((N, EMBED), jnp.float32),
    )


_mlp_first = _make_mlp(first=True)
_mlp_res = _make_mlp(first=False)


def kernel(x, edge_index, edge_attr,
           l0_We, l0_be, l0_W1, l0_b1, l0_W2, l0_b2,
           l1_We, l1_be, l1_W1, l1_b1, l1_W2, l1_b2,
           l2_We, l2_be, l2_W1, l2_b1, l2_W2, l2_b2):
    rs = jax.random.normal(jax.random.key(42), (N, 1), dtype=x.dtype)

    src = edge_index[0]
    dst = edge_index[1]
    dst3 = dst.reshape(NS, 1, EPT)
    attr = edge_attr[:, 0]
    zrows = jnp.zeros((RPT, Dh), jnp.float32)

    WeH = l0_We[0, :256].reshape(2, Dh)
    beH = l0_be[:256].reshape(2, Dh)
    wx = jnp.full((16,), l0_We[0, 256], jnp.float32)
    bx = jnp.full((16,), l0_be[256], jnp.float32)
    rs_pad = jnp.concatenate([rs[:, 0], jnp.zeros((NACC - N,), jnp.float32)])

    aggr = _sc_aggr(x.reshape(2 * N, Dh), src, dst3, attr, WeH, beH, zrows)
    aggr_x = _sc_extra(rs_pad, src, dst, attr, wx, bx)
    xcol = rs + aggr_x[:N, None]

    W1p = jnp.concatenate(
        [l0_W1, jnp.zeros((7, HIDDEN), l0_W1.dtype)]).astype(jnp.bfloat16)
    xprev = _mlp_first(x, aggr[0], aggr[1],
                       W1p, l0_b1.reshape(1, HIDDEN),
                       l0_W2.astype(jnp.bfloat16), l0_b2.reshape(1, EMBED),
                       xcol)

    for (We, be, W1, b1v, W2, b2v) in (
            (l1_We, l1_be, l1_W1, l1_b1, l1_W2, l1_b2),
            (l2_We, l2_be, l2_W1, l2_b1, l2_W2, l2_b2)):
        WeH = We[0].reshape(2, Dh)
        beH = be.reshape(2, Dh)
        aggr = _sc_aggr(xprev.reshape(2 * N, Dh), src, dst3, attr,
                        WeH, beH, zrows)
        xprev = _mlp_res(xprev, aggr[0], aggr[1],
                         W1.astype(jnp.bfloat16), b1v.reshape(1, HIDDEN),
                         W2.astype(jnp.bfloat16), b2v.reshape(1, EMBED),
                         xprev)
    return xprev


# speedup vs baseline: 1.7963x; 1.0014x over previous
import functools

import jax
import jax.numpy as jnp
from jax import lax
from jax.experimental import pallas as pl
from jax.experimental.pallas import tpu as pltpu
from jax.experimental.pallas import tpu_sc as plsc

N = 10000
E = 160000
HIDDEN = 4096
EMBED = 256
Dh = 128

NS = 16
EPT = E // NS
B = 80
NCHUNK = EPT // B
NACC = 10240
RPT = NACC // NS

_mesh = plsc.VectorSubcoreMesh(core_axis_name="c", subcore_axis_name="s",
                               num_cores=2, num_subcores=NS)


@functools.partial(
    pl.kernel,
    out_type=jax.ShapeDtypeStruct((2, NACC, Dh), jnp.float32),
    mesh=_mesh,
    scratch_types=[
        [pltpu.VMEM((B,), jnp.int32)] * 2,
        [pltpu.VMEM((B,), jnp.int32)] * 2,
        [pltpu.VMEM((B,), jnp.float32)] * 2,
        [pltpu.VMEM((B, Dh), jnp.float32)] * 2,
        [pltpu.VMEM((B,), jnp.int32)] * 2,
        pltpu.VMEM((1, EPT), jnp.int32),
        pltpu.VMEM((Dh,), jnp.float32),
        pltpu.VMEM((Dh,), jnp.float32),
        pltpu.VMEM_SHARED((NACC, Dh), jnp.float32),
        [pltpu.SemaphoreType.DMA] * 2,
        [pltpu.SemaphoreType.DMA] * 2,
        [pltpu.SemaphoreType.DMA] * 2,
    ],
)
def _sc_aggr(table, src, dst3, attr, WeH, beH, zrows, out,
             gi, sb, ab, rows, di, dall, wev, bev, aggr,
             sem_i, sem_g, sem_s):
    c = lax.axis_index("c")
    s = lax.axis_index("s")
    pltpu.sync_copy(WeH.at[c], wev)
    pltpu.sync_copy(beH.at[c], bev)
    pltpu.sync_copy(zrows, aggr.at[pl.ds(s * RPT, RPT)])
    plsc.subcore_barrier()
    pltpu.sync_copy(dst3.at[s], dall)

    we = [wev[pl.ds(16 * j, 16)] for j in range(Dh // 16)]
    be = [bev[pl.ds(16 * j, 16)] for j in range(Dh // 16)]

    ebase = s * EPT

    def fire_in(k, b):
        eb = ebase + k * B
        pltpu.async_copy(src.at[pl.ds(eb, B)], sb[b], sem_i[b])
        pltpu.async_copy(attr.at[pl.ds(eb, B)], ab[b], sem_i[b])

    def wait_in(k, b):
        eb = ebase + k * B
        pltpu.make_async_copy(src.at[pl.ds(eb, B)], sb[b], sem_i[b]).wait()
        pltpu.make_async_copy(attr.at[pl.ds(eb, B)], ab[b], sem_i[b]).wait()

    def transform(b):
        for j in range(B // 16):
            v = sb[b][pl.ds(16 * j, 16)]
            gi[b][pl.ds(16 * j, 16)] = v * 2 + c

    def fire_gather(b):
        pltpu.async_copy(table.at[gi[b]], rows[b], sem_g[b])

    def wait_gather(b):
        pltpu.make_async_copy(table.at[gi[b]], rows[b], sem_g[b]).wait()

    def fire_scatter(b):
        pltpu.async_copy(rows[b], aggr.at[di[b]], sem_s[b], add=True)

    def wait_scatter(b):
        pltpu.make_async_copy(rows[b], aggr.at[di[b]], sem_s[b]).wait()

    def fill_di(k, b):
        for j in range(B // 16):
            di[b][pl.ds(16 * j, 16)] = dall[0, pl.ds(k * B + 16 * j, 16)]

    def compute(b):
        def grp(g, _):
            a_vec = ab[b][pl.ds(16 * g, 16)]
            for i in range(16):
                a = a_vec[i]
                e = 16 * g + i
                for j in range(Dh // 16):
                    x = rows[b][e, pl.ds(16 * j, 16)]
                    rows[b][e, pl.ds(16 * j, 16)] = jnp.maximum(
                        x + (a * we[j] + be[j]), 0.0)
            return 0

        lax.fori_loop(0, B // 16, grp, 0)

    fire_in(0, 0)
    fire_in(1, 1)
    wait_in(0, 0)
    transform(0)
    fire_gather(0)

    def step(k, b):
        bo = 1 - b
        wait_gather(b)
        wait_in(k + 1, bo)
        transform(bo)

        @pl.when(k >= 1)
        def _():
            wait_scatter(bo)

        fire_gather(bo)
        fill_di(k, b)
        compute(b)
        fire_scatter(b)

        @pl.when(k + 2 < NCHUNK)
        def _():
            fire_in(k + 2, b)

    def outer(K, _):
        step(2 * K, 0)
        step(2 * K + 1, 1)
        return 0

    lax.fori_loop(0, (NCHUNK - 1) // 2, outer, 0)

    wait_gather(0)
    fill_di(NCHUNK - 1, 0)
    compute(0)
    fire_scatter(0)
    wait_scatter(1)
    wait_scatter(0)

    plsc.subcore_barrier()
    pltpu.sync_copy(aggr.at[pl.ds(s * RPT, RPT)],
                    out.at[c, pl.ds(s * RPT, RPT)])


@functools.partial(
    pl.kernel,
    out_type=jax.ShapeDtypeStruct((NACC,), jnp.float32),
    mesh=_mesh,
    scratch_types=[
        pltpu.VMEM((EPT,), jnp.int32),
        pltpu.VMEM((EPT,), jnp.int32),
        pltpu.VMEM((EPT,), jnp.float32),
        pltpu.VMEM((NACC,), jnp.float32),
        pltpu.VMEM((NACC,), jnp.float32),
        pltpu.VMEM((16, RPT), jnp.float32),
        pltpu.VMEM((RPT,), jnp.float32),
        pltpu.VMEM((16,), jnp.float32),
        pltpu.VMEM((16,), jnp.float32),
        pltpu.VMEM_SHARED((16, NACC), jnp.float32),
        pltpu.SemaphoreType.DMA,
    ],
    compiler_params=pltpu.CompilerParams(needs_layout_passes=False),
)
def _sc_extra(rs, src, dst, attr, wxH, bxH, out,
              gi, di, av, rsv, acc, red16, red, wxv, bxv, acc_sh, sem):
    c = lax.axis_index("c")
    s = lax.axis_index("s")

    @pl.when(c == 0)
    def _():
        pltpu.sync_copy(rs, rsv)
        pltpu.sync_copy(wxH, wxv)
        pltpu.sync_copy(bxH, bxv)
        pltpu.sync_copy(src.at[pl.ds(s * EPT, EPT)], gi)
        pltpu.sync_copy(dst.at[pl.ds(s * EPT, EPT)], di)
        pltpu.sync_copy(attr.at[pl.ds(s * EPT, EPT)], av)
        wx = wxv[...]
        bx = bxv[...]
        zero = jnp.zeros((16,), jnp.float32)
        def z(j, _):
            acc[pl.ds(16 * j, 16)] = zero
            return 0
        lax.fori_loop(0, NACC // 16, z, 0)

        def group(g, _):
            s16 = gi[pl.ds(16 * g, 16)]
            d16 = di[pl.ds(16 * g, 16)]
            a16 = av[pl.ds(16 * g, 16)]
            vals = plsc.load_gather(rsv, [s16])
            msg = jnp.maximum(vals + (a16 * wx + bx), 0.0)
            plsc.addupdate_scatter(acc, [d16], msg)
            return 0

        lax.fori_loop(0, EPT // 16, group, 0)

        pltpu.sync_copy(acc, acc_sh.at[s])
        plsc.subcore_barrier()
        pltpu.sync_copy(acc_sh.at[:, pl.ds(s * RPT, RPT)], red16)

        def radd(j, _):
            t = red16[0, pl.ds(16 * j, 16)]
            for i in range(1, 16):
                t = t + red16[i, pl.ds(16 * j, 16)]
            red[pl.ds(16 * j, 16)] = t
            return 0

        lax.fori_loop(0, RPT // 16, radd, 0)
        pltpu.sync_copy(red, out.at[pl.ds(s * RPT, RPT)])


def _make_mlp(first):
    R = 1000
    G = N // R
    D = 256
    KD = 264 if first else 256

    def body(h_ref, a0_ref, a1_ref, w1_ref, b1_ref, w2_ref, b2_ref,
             *rest):
        if first:
            xcol_ref, out_ref = rest
        else:
            xp_ref, out_ref = rest
        za = jnp.concatenate([a0_ref[...], a1_ref[...]], axis=1)
        z = (h_ref[...] + za).astype(jnp.bfloat16)
        if first:
            xcb = jnp.concatenate(
                [xcol_ref[...], jnp.zeros((R, 7), jnp.float32)],
                axis=1).astype(jnp.bfloat16)
            z = jnp.concatenate([z, xcb], axis=1)
        hh = jnp.dot(z, w1_ref[...], preferred_element_type=jnp.float32)
        hh = jnp.maximum(hh + b1_ref[...], 0.0).astype(jnp.bfloat16)
        o = jnp.dot(hh, w2_ref[...], preferred_element_type=jnp.float32) + b2_ref[...]
        if not first:
            o = (jnp.where(o > 0, o, 0.01 * o) + xp_ref[...]) * 0.5
        out_ref[...] = o

    in_specs = [
        pl.BlockSpec((R, D), lambda i: (i, 0)),
        pl.BlockSpec((R, Dh), lambda i: (i, 0)),
        pl.BlockSpec((R, Dh), lambda i: (i, 0)),
        pl.BlockSpec((KD, HIDDEN), lambda i: (0, 0)),
        pl.BlockSpec((1, HIDDEN), lambda i: (0, 0)),
        pl.BlockSpec((HIDDEN, EMBED), lambda i: (0, 0)),
        pl.BlockSpec((1, EMBED), lambda i: (0, 0)),
    ]
    if first:
        in_specs.append(pl.BlockSpec((R, 1), lambda i: (i, 0)))
    else:
        in_specs.append(pl.BlockSpec((R, EMBED), lambda i: (i, 0)))

    return pl.pallas_call(
        body,
        grid=(G,),
        in_specs=in_specs,
        out_specs=pl.BlockSpec((R, EMBED), lambda i: (i, 0)),
        out_shape=jax.ShapeDtypeStruct((N, EMBED), jnp.float32),
    )


_mlp_first = _make_mlp(first=True)
_mlp_res = _make_mlp(first=False)


def kernel(x, edge_index, edge_attr,
           l0_We, l0_be, l0_W1, l0_b1, l0_W2, l0_b2,
           l1_We, l1_be, l1_W1, l1_b1, l1_W2, l1_b2,
           l2_We, l2_be, l2_W1, l2_b1, l2_W2, l2_b2):
    rs = jax.random.normal(jax.random.key(42), (N, 1), dtype=x.dtype)

    src = edge_index[0]
    dst = edge_index[1]
    dst3 = dst.reshape(NS, 1, EPT)
    attr = edge_attr[:, 0]
    zrows = jnp.zeros((RPT, Dh), jnp.float32)

    WeH = l0_We[0, :256].reshape(2, Dh)
    beH = l0_be[:256].reshape(2, Dh)
    wx = jnp.full((16,), l0_We[0, 256], jnp.float32)
    bx = jnp.full((16,), l0_be[256], jnp.float32)
    rs_pad = jnp.concatenate([rs[:, 0], jnp.zeros((NACC - N,), jnp.float32)])

    aggr = _sc_aggr(x.reshape(2 * N, Dh), src, dst3, attr, WeH, beH, zrows)
    aggr_x = _sc_extra(rs_pad, src, dst, attr, wx, bx)
    xcol = rs + aggr_x[:N, None]

    W1p = jnp.concatenate(
        [l0_W1, jnp.zeros((7, HIDDEN), l0_W1.dtype)]).astype(jnp.bfloat16)
    xprev = _mlp_first(x, aggr[0], aggr[1],
                       W1p, l0_b1.reshape(1, HIDDEN),
                       l0_W2.astype(jnp.bfloat16), l0_b2.reshape(1, EMBED),
                       xcol)

    for (We, be, W1, b1v, W2, b2v) in (
            (l1_We, l1_be, l1_W1, l1_b1, l1_W2, l1_b2),
            (l2_We, l2_be, l2_W1, l2_b1, l2_W2, l2_b2)):
        WeH = We[0].reshape(2, Dh)
        beH = be.reshape(2, Dh)
        aggr = _sc_aggr(xprev.reshape(2 * N, Dh), src, dst3, attr,
                        WeH, beH, zrows)
        xprev = _mlp_res(xprev, aggr[0], aggr[1],
                         W1.astype(jnp.bfloat16), b1v.reshape(1, HIDDEN),
                         W2.astype(jnp.bfloat16), b2v.reshape(1, EMBED),
                         xprev)
    return xprev
